# Initial kernel scaffold; baseline (speedup 1.0000x reference)
import jax
import jax.numpy as jnp
from jax.experimental import pallas as pl


def kernel(z, emb):
    raise NotImplementedError("write your pallas kernel here")


# speedup vs baseline: 1.6748x; 1.6748x over previous
import jax
import jax.numpy as jnp
from jax.experimental import pallas as pl

_K = 1024
_ROWS_PER_BLOCK = 1024


def _vq_block(tmp_ref, emb_ref, q_ref, ste_ref, idx_ref):
    tmp = tmp_ref[...]
    emb = emb_ref[...]
    s2 = jnp.sum(tmp * tmp, axis=1, keepdims=True)
    e2 = jnp.sum(emb * emb, axis=1)[None, :]
    ab = jax.lax.dot_general(
        tmp, emb, (((1,), (1,)), ((), ())),
        preferred_element_type=jnp.float32,
        precision=jax.lax.Precision.DEFAULT)
    d = (s2 - 2.0 * ab) + e2
    mind = jnp.min(d, axis=1, keepdims=True)
    iota = jax.lax.broadcasted_iota(jnp.int32, d.shape, 1)
    idx = jnp.min(jnp.where(d == mind, iota, _K), axis=1)
    oh = (iota == idx[:, None]).astype(jnp.float32)
    q = jax.lax.dot_general(
        emb, oh, (((0,), (1,)), ((), ())),
        preferred_element_type=jnp.float32)
    q_ref[0] = q
    ste_ref[0] = q
    idx_ref[0, 0] = idx


def kernel(z, emb):
    n, c, h, w = z.shape
    k = emb.shape[0]
    s_total = n * h * w
    tmp = jnp.transpose(z, (0, 2, 3, 1)).reshape(s_total, c)
    blk = _ROWS_PER_BLOCK
    nblk = s_total // blk
    q, ste, idx = pl.pallas_call(
        _vq_block,
        grid=(nblk,),
        in_specs=[
            pl.BlockSpec((blk, c), lambda b: (b, 0)),
            pl.BlockSpec((k, c), lambda b: (0, 0)),
        ],
        out_specs=[
            pl.BlockSpec((1, c, blk), lambda b: (b, 0, 0)),
            pl.BlockSpec((1, c, blk), lambda b: (b, 0, 0)),
            pl.BlockSpec((1, 1, blk), lambda b: (b, 0, 0)),
        ],
        out_shape=[
            jax.ShapeDtypeStruct((nblk, c, blk), jnp.float32),
            jax.ShapeDtypeStruct((nblk, c, blk), jnp.float32),
            jax.ShapeDtypeStruct((nblk, 1, blk), jnp.int32),
        ],
    )(tmp, emb)
    quantized = q.reshape(n, c, h, w)
    ste_out = ste.reshape(n, c, h, w)
    indxs = idx.reshape(n, h, w)
    return (quantized, ste_out, indxs)
